# Initial kernel scaffold; baseline (speedup 1.0000x reference)
import jax
import jax.numpy as jnp
from jax.experimental import pallas as pl


def kernel(x, edge_index, W1, b1, W2, b2, W3, b3, W4, b4, W5, b5, Wc, bc):
    raise NotImplementedError("write your pallas kernel here")


# speedup vs baseline: 14.1524x; 14.1524x over previous
import functools

import jax
import jax.numpy as jnp
from jax import lax
from jax.experimental import pallas as pl
from jax.experimental.pallas import tpu as pltpu
from jax.experimental.pallas import tpu_sc as plsc

N = 10000
N_PAD = 10240
E = 320000
H = 128
HH = H // 2
NC = 2
NS = 16
NW = NC * NS
EPW = E // NW
NCHUNK = 100
CHUNK = EPW // NCHUNK
RPT = N_PAD // NS

_R = 1024

_MESH = plsc.VectorSubcoreMesh(core_axis_name="c", subcore_axis_name="s")


@functools.partial(
    pl.kernel,
    out_type=jax.ShapeDtypeStruct((NC, N_PAD, H), jnp.float32),
    mesh=_MESH,
    scratch_types=[
        pltpu.VMEM((NCHUNK, CHUNK), jnp.int32),
        pltpu.VMEM((NCHUNK, CHUNK), jnp.int32),
        pltpu.VMEM((CHUNK, H), jnp.float32),
        pltpu.VMEM_SHARED((N_PAD, H), jnp.float32),
        pltpu.SemaphoreType.DMA,
    ],
)
def _agg(hp, src, dst, zeros, out, src_v, dst_v, rb0, acc, sem0):
  cid = lax.axis_index("c")
  sid = lax.axis_index("s")
  wid = sid * NC + cid
  pltpu.sync_copy(src.at[wid], src_v)
  pltpu.sync_copy(dst.at[wid], dst_v)
  pltpu.sync_copy(zeros, acc.at[pl.ds(sid * RPT, RPT)])
  plsc.subcore_barrier()

  def gather(c, buf, sem):
    return pltpu.make_async_copy(hp.at[src_v.at[c]], buf, sem)

  def scat(c, buf):
    pltpu.sync_copy(buf, acc.at[dst_v.at[c]], add=True)

  def body(i, carry):
    g0 = gather(i, rb0, sem0)
    g0.start()
    g0.wait()
    scat(i, rb0)
    return carry

  lax.fori_loop(0, NCHUNK, body, 0)
  plsc.subcore_barrier()
  pltpu.sync_copy(acc.at[pl.ds(sid * RPT, RPT)],
                  out.at[cid, pl.ds(sid * RPT, RPT)])


@functools.partial(
    pl.kernel,
    out_type=jax.ShapeDtypeStruct((NC, N_PAD, H), jnp.float32),
    mesh=_MESH,
    scratch_types=[
        pltpu.VMEM((NCHUNK, CHUNK), jnp.int32),
        pltpu.VMEM((CHUNK, H), jnp.float32),
        pltpu.VMEM_SHARED((N_PAD, H), jnp.float32),
    ],
)
def _deg(dst, ones, zeros, out, dst_v, ones_v, acc):
  cid = lax.axis_index("c")
  sid = lax.axis_index("s")
  wid = sid * NC + cid
  pltpu.sync_copy(zeros, acc.at[pl.ds(sid * RPT, RPT)])
  pltpu.sync_copy(dst.at[wid], dst_v)
  pltpu.sync_copy(ones, ones_v)
  plsc.subcore_barrier()

  def body(c, carry):
    pltpu.sync_copy(ones_v, acc.at[dst_v.at[c]], add=True)
    return carry

  lax.fori_loop(0, NCHUNK, body, 0)
  plsc.subcore_barrier()
  pltpu.sync_copy(acc.at[pl.ds(sid * RPT, RPT)],
                  out.at[cid, pl.ds(sid * RPT, RPT)])


def _elu(v):
  return jnp.where(v > 0, v, jnp.exp(v) - 1.0)


_SPLIT_SPEC = pl.BlockSpec((_R, H), lambda i: (i, 0))
_PARTS_SPEC = pl.BlockSpec((NC, _R, H), lambda i: (0, i, 0))
_COL_SPEC = pl.BlockSpec((_R, 1), lambda i: (i, 0))


def _tc_first(x, W1, degp):
  D = x.shape[1]

  def body(x_ref, w_ref, d_ref, hp_ref, dinv_ref):
    deg = d_ref[0][:, 0:1] + d_ref[1][:, 0:1] + 1.0
    dinv = lax.rsqrt(deg)
    hp_ref[...] = (x_ref[...] @ w_ref[...]) * dinv
    dinv_ref[...] = dinv

  return pl.pallas_call(
      body,
      grid=(N_PAD // _R,),
      in_specs=[
          pl.BlockSpec((_R, D), lambda i: (i, 0)),
          pl.BlockSpec((D, H), lambda i: (0, 0)),
          pl.BlockSpec((2, _R, H), lambda i: (0, i, 0)),
      ],
      out_specs=[_SPLIT_SPEC, _COL_SPEC],
      out_shape=[
          jax.ShapeDtypeStruct((N_PAD, H), jnp.float32),
          jax.ShapeDtypeStruct((N_PAD, 1), jnp.float32),
      ],
  )(x, W1, degp)


def _tc_mid(parts, hp, dinv, b, Wn):
  def body(p_ref, hp_ref, dinv_ref, b_ref, w_ref, o_ref):
    dinv = dinv_ref[...]
    z = _elu(dinv * (p_ref[0] + p_ref[1] + hp_ref[...]) + b_ref[...])
    o_ref[...] = (z @ w_ref[...]) * dinv

  return pl.pallas_call(
      body,
      grid=(N_PAD // _R,),
      in_specs=[
          _PARTS_SPEC,
          _SPLIT_SPEC,
          _COL_SPEC,
          pl.BlockSpec((1, H), lambda i: (0, 0)),
          pl.BlockSpec((H, H), lambda i: (0, 0)),
      ],
      out_specs=_SPLIT_SPEC,
      out_shape=jax.ShapeDtypeStruct((N_PAD, H), jnp.float32),
  )(parts, hp, dinv, b, Wn)


def _tc_final(parts, hp, dinv, b, Wc, bc):
  def body(p_ref, hp_ref, dinv_ref, b_ref, w_ref, bc_ref, o_ref):
    z = _elu(dinv_ref[...] * (p_ref[0] + p_ref[1] + hp_ref[...]) + b_ref[...])
    o_ref[...] = z @ w_ref[...] + bc_ref[...]

  return pl.pallas_call(
      body,
      grid=(N_PAD // _R,),
      in_specs=[
          _PARTS_SPEC,
          _SPLIT_SPEC,
          _COL_SPEC,
          pl.BlockSpec((1, H), lambda i: (0, 0)),
          pl.BlockSpec((H, 1), lambda i: (0, 0)),
          pl.BlockSpec((1, 1), lambda i: (0, 0)),
      ],
      out_specs=pl.BlockSpec((_R, 1), lambda i: (i, 0)),
      out_shape=jax.ShapeDtypeStruct((N_PAD, 1), jnp.float32),
  )(parts, hp, dinv, b, Wc, bc)


def kernel(x, edge_index, W1, b1, W2, b2, W3, b3, W4, b4, W5, b5, Wc, bc):
  f32 = jnp.float32
  src = edge_index[0].reshape(NW, NCHUNK, CHUNK)
  dst = edge_index[1].reshape(NW, NCHUNK, CHUNK)
  xp = jnp.pad(x, ((0, N_PAD - N), (0, 0)))
  zh = jnp.zeros((RPT, H), f32)
  z16 = jnp.zeros((RPT, H), f32)
  ones16 = jnp.ones((CHUNK, H), f32)

  W4p = jnp.pad(W4, ((0, 0), (0, H - 11)))
  b4p = jnp.pad(b4, (0, H - 11)).reshape(1, H)
  W5p = jnp.pad(W5, ((0, H - 11), (0, H - 11)))
  b5p = jnp.pad(b5, (0, H - 11)).reshape(1, H)
  Wcp = jnp.pad(Wc, ((0, H - 11), (0, 0)))

  degp = _deg(dst, ones16, z16)
  h1, dinv = _tc_first(xp, W1, degp)
  p1 = _agg(h1, src, dst, zh)
  h2 = _tc_mid(p1, h1, dinv, b1.reshape(1, H), W2)
  p2 = _agg(h2, src, dst, zh)
  h3 = _tc_mid(p2, h2, dinv, b2.reshape(1, H), W3)
  p3 = _agg(h3, src, dst, zh)
  h4 = _tc_mid(p3, h3, dinv, b3.reshape(1, H), W4p)
  p4 = _agg(h4, src, dst, zh)
  h5 = _tc_mid(p4, h4, dinv, b4p, W5p)
  p5 = _agg(h5, src, dst, zh)
  out = _tc_final(p5, h5, dinv, b5p, Wcp, bc.reshape(1, 1))
  return out[:N]
